# baseline (speedup 1.0000x reference)
import jax
import jax.numpy as jnp
from jax import lax
from jax.experimental import pallas as pl
from jax.experimental.pallas import tpu as pltpu
from jax.experimental.pallas import tpu_sc as plsc

L = 200
D = 64
B = 4096
T = B * L
NC, NS = 2, 16
NW = NC * NS
C = 400
G = 4
GI = C // G
N_CHUNKS = T // C
CPW = N_CHUNKS // NW


def _emb_body(x_hbm, wt_hbm, pos_hbm, out_hbm, pos_v, idx_v, rows_v, sem):
    wid = lax.axis_index("s") * NC + lax.axis_index("c")
    pltpu.sync_copy(pos_hbm, pos_v)

    def chunk_body(ci, carry):
        c = wid * CPW + ci
        pltpu.sync_copy(x_hbm.at[c], idx_v)
        handles = [
            pltpu.async_copy(
                wt_hbm.at[idx_v.at[g]], rows_v.at[pl.ds(g * GI, GI)], sem
            )
            for g in range(G)
        ]
        for h in handles:
            h.wait()

        def add_body(i, carry2):
            for j in range(D // 16):
                sl = pl.ds(j * 16, 16)
                pv = pos_v[i, sl]
                rows_v[i, sl] += pv
                rows_v[L + i, sl] += pv
            return carry2

        lax.fori_loop(0, L, add_body, 0)
        pltpu.sync_copy(rows_v, out_hbm.at[pl.ds(c * C, C)])
        return carry

    lax.fori_loop(0, CPW, chunk_body, 0)


def kernel(x, word_table, pos_table):
    x3 = x.reshape(N_CHUNKS, G, GI).astype(jnp.int32)
    mesh = plsc.VectorSubcoreMesh(core_axis_name="c", subcore_axis_name="s")
    out = pl.kernel(
        _emb_body,
        out_type=jax.ShapeDtypeStruct((T, D), jnp.float32),
        mesh=mesh,
        compiler_params=pltpu.CompilerParams(use_tc_tiling_on_sc=False),
        scratch_types=[
            pltpu.VMEM((L, D), jnp.float32),
            pltpu.VMEM((G, GI), jnp.int32),
            pltpu.VMEM((C, D), jnp.float32),
            pltpu.SemaphoreType.DMA,
        ],
    )(x3, word_table, pos_table)
    return out.reshape(B, L, D)


# speedup vs baseline: 1.1279x; 1.1279x over previous
import jax
import jax.numpy as jnp
from jax import lax
from jax.experimental import pallas as pl
from jax.experimental.pallas import tpu as pltpu
from jax.experimental.pallas import tpu_sc as plsc

L = 200
D = 64
B = 4096
T = B * L
NC, NS = 2, 16
NW = NC * NS
C = 400
G = 4
GI = C // G
N_CHUNKS = T // C
CPW = N_CHUNKS // NW
NBUF = 4
NOUTER = CPW // NBUF


def _emb_body(x_hbm, wt_hbm, pos_hbm, out_hbm, pos_v, idx_v, rows_v,
              sg0, sg1, sg2, sg3, ss0, ss1, ss2, ss3, si0, si1, si2, si3):
    sg = (sg0, sg1, sg2, sg3)
    ss = (ss0, ss1, ss2, ss3)
    si = (si0, si1, si2, si3)
    wid = lax.axis_index("s") * NC + lax.axis_index("c")
    base = wid * CPW
    pltpu.sync_copy(pos_hbm, pos_v)

    def fire_idx(p, bp):
        pltpu.async_copy(x_hbm.at[base + p], idx_v.at[bp], si[bp])

    def wait_idx(bp):
        pltpu.make_async_copy(x_hbm.at[0], idx_v.at[bp], si[bp]).wait()

    def fire_gathers(bp):
        for g in range(G):
            pltpu.async_copy(
                wt_hbm.at[idx_v.at[bp, g]],
                rows_v.at[bp].at[pl.ds(g * GI, GI)],
                sg[bp],
            )

    def wait_gathers(bp):
        pltpu.make_async_copy(wt_hbm.at[pl.ds(0, C)], rows_v.at[bp], sg[bp]).wait()

    def fire_store(c, bp):
        pltpu.async_copy(rows_v.at[bp], out_hbm.at[pl.ds((base + c) * C, C)], ss[bp])

    def wait_store(bp):
        pltpu.make_async_copy(rows_v.at[bp], out_hbm.at[pl.ds(0, C)], ss[bp]).wait()

    fire_idx(0, 0)
    wait_idx(0)
    fire_gathers(0)
    fire_idx(1, 1)

    def outer(ci, carry):
        for b in range(NBUF):
            c = ci * NBUF + b
            bp = (b + 1) % NBUF

            if b < NBUF - 1:
                wait_idx(bp)

                @pl.when(ci >= 1)
                def _():
                    wait_store(bp)

                fire_gathers(bp)
                if b < NBUF - 2:
                    fire_idx(c + 2, (b + 2) % NBUF)
                else:
                    @pl.when(ci < NOUTER - 1)
                    def _():
                        fire_idx(c + 2, (b + 2) % NBUF)
            else:
                @pl.when(ci < NOUTER - 1)
                def _():
                    wait_idx(bp)
                    wait_store(bp)
                    fire_gathers(bp)
                    fire_idx(c + 2, (b + 2) % NBUF)

            wait_gathers(b)

            def add_body(i, carry2):
                for j in range(D // 16):
                    sl = pl.ds(j * 16, 16)
                    pv = pos_v[i, sl]
                    rows_v[b, i, sl] += pv
                    rows_v[b, L + i, sl] += pv
                return carry2

            lax.fori_loop(0, L, add_body, 0)
            fire_store(c, b)
        return carry

    lax.fori_loop(0, NOUTER, outer, 0)
    for b in range(NBUF):
        wait_store(b)


def kernel(x, word_table, pos_table):
    x3 = x.reshape(N_CHUNKS, G, GI).astype(jnp.int32)
    mesh = plsc.VectorSubcoreMesh(core_axis_name="c", subcore_axis_name="s")
    out = pl.kernel(
        _emb_body,
        out_type=jax.ShapeDtypeStruct((T, D), jnp.float32),
        mesh=mesh,
        compiler_params=pltpu.CompilerParams(use_tc_tiling_on_sc=False),
        scratch_types=[
            pltpu.VMEM((L, D), jnp.float32),
            pltpu.VMEM((NBUF, G, GI), jnp.int32),
            pltpu.VMEM((NBUF, C, D), jnp.float32),
        ] + [pltpu.SemaphoreType.DMA] * 12,
    )(x3, word_table, pos_table)
    return out.reshape(B, L, D)
